# Initial kernel scaffold; baseline (speedup 1.0000x reference)
import jax
import jax.numpy as jnp
from jax.experimental import pallas as pl


def kernel(x, sem_labels, table, bbox):
    raise NotImplementedError("write your pallas kernel here")


# speedup vs baseline: 1.4552x; 1.4552x over previous
import functools

import jax
import jax.numpy as jnp
from jax import lax
from jax.experimental import pallas as pl
from jax.experimental.pallas import tpu as pltpu
from jax.experimental.pallas import tpu_sc as plsc

_NUM_CLASSES = 150
_D = 768


def _sc_concat_gather(x2, labels, table):
    R = x2.shape[0]
    info = plsc.get_sparse_core_info()
    NC, NS = info.num_cores, info.num_subcores
    NW = NC * NS
    assert R % NW == 0
    b_per_w = R // NW
    CH = 64
    n_ch = b_per_w // CH
    assert b_per_w % CH == 0

    mesh = plsc.VectorSubcoreMesh(core_axis_name="c", subcore_axis_name="s")

    @functools.partial(
        pl.kernel,
        mesh=mesh,
        out_type=jax.ShapeDtypeStruct((R, 2 * _D), jnp.float32),
        scratch_types=[
            pltpu.VMEM((b_per_w,), jnp.int32),
            pltpu.VMEM((CH, _D), jnp.float32),
            pltpu.VMEM((CH, _D), jnp.float32),
            pltpu.SemaphoreType.DMA,
        ],
    )
    def k(x_hbm, lab_hbm, tab_hbm, out_hbm, idx_v, xbuf, ebuf, sem):
        wid = lax.axis_index("s") * NC + lax.axis_index("c")
        base = wid * b_per_w
        pltpu.sync_copy(lab_hbm.at[pl.ds(base, b_per_w)], idx_v)
        for c in range(n_ch):
            r0 = base + c * CH
            pltpu.sync_copy(x_hbm.at[pl.ds(r0, CH), :], xbuf)
            pltpu.sync_copy(xbuf, out_hbm.at[pl.ds(r0, CH), pl.ds(0, _D)])
            pltpu.async_copy(
                tab_hbm.at[idx_v.at[pl.ds(c * CH, CH)]], ebuf, sem
            ).wait()
            pltpu.sync_copy(ebuf, out_hbm.at[pl.ds(r0, CH), pl.ds(_D, _D)])

    return k(x2, labels, table)


def kernel(x, sem_labels, table, bbox):
    B, N, C = x.shape
    x2 = x.reshape(B * N, C)
    labels = sem_labels.reshape(B * N).astype(jnp.int32)
    out = _sc_concat_gather(x2, labels, table)
    return out.reshape(B, N, 2 * C)


# speedup vs baseline: 1.4911x; 1.0246x over previous
import functools

import jax
import jax.numpy as jnp
from jax import lax
from jax.experimental import pallas as pl
from jax.experimental.pallas import tpu as pltpu
from jax.experimental.pallas import tpu_sc as plsc

_NUM_CLASSES = 150
_D = 768


def _sc_concat_gather(x2, labels, table):
    R = x2.shape[0]
    info = plsc.get_sparse_core_info()
    NC, NS = info.num_cores, info.num_subcores
    NW = NC * NS
    assert R % NW == 0
    b_per_w = R // NW
    CH = 32
    n_ch = b_per_w // CH
    assert b_per_w % CH == 0

    mesh = plsc.VectorSubcoreMesh(core_axis_name="c", subcore_axis_name="s")

    @functools.partial(
        pl.kernel,
        mesh=mesh,
        out_type=jax.ShapeDtypeStruct((R, 2 * _D), jnp.float32),
        scratch_types=[
            pltpu.VMEM((b_per_w,), jnp.int32),
            pltpu.VMEM((CH, 2 * _D), jnp.float32),
            pltpu.VMEM((CH, 2 * _D), jnp.float32),
            pltpu.SemaphoreType.DMA,
            pltpu.SemaphoreType.DMA,
            pltpu.SemaphoreType.DMA,
            pltpu.SemaphoreType.DMA,
        ],
    )
    def k(x_hbm, lab_hbm, tab_hbm, out_hbm, idx_v, b0, b1, is0, is1, os0, os1):
        bufs = (b0, b1)
        isems = (is0, is1)
        osems = (os0, os1)
        wid = lax.axis_index("s") * NC + lax.axis_index("c")
        base = wid * b_per_w
        pltpu.sync_copy(lab_hbm.at[pl.ds(base, b_per_w)], idx_v)
        prev_out = [None, None]
        for c in range(n_ch):
            i = c & 1
            r0 = base + c * CH
            if prev_out[i] is not None:
                prev_out[i].wait()
            x_cp = pltpu.make_async_copy(
                x_hbm.at[pl.ds(r0, CH), :], bufs[i].at[:, pl.ds(0, _D)],
                isems[i])
            e_cp = pltpu.make_async_copy(
                tab_hbm.at[idx_v.at[pl.ds(c * CH, CH)]],
                bufs[i].at[:, pl.ds(_D, _D)], isems[i])
            x_cp.start()
            e_cp.start()
            x_cp.wait()
            e_cp.wait()
            o_cp = pltpu.make_async_copy(
                bufs[i], out_hbm.at[pl.ds(r0, CH), :], osems[i])
            o_cp.start()
            prev_out[i] = o_cp
        prev_out[0].wait()
        prev_out[1].wait()

    return k(x2, labels, table)


def kernel(x, sem_labels, table, bbox):
    B, N, C = x.shape
    x2 = x.reshape(B * N, C)
    labels = sem_labels.reshape(B * N).astype(jnp.int32)
    out = _sc_concat_gather(x2, labels, table)
    return out.reshape(B, N, 2 * C)


# speedup vs baseline: 2.1920x; 1.4701x over previous
import functools

import jax
import jax.numpy as jnp
from jax import lax
from jax.experimental import pallas as pl
from jax.experimental.pallas import tpu as pltpu

_NUM_CLASSES = 150
_D = 768
_BM = 256


def _tc_body(lab_ref, x_ref, tab_ref, out_ref):
    out_ref[:, : _D] = x_ref[...]
    labels = lab_ref[0, 0, :]
    onehot = (labels[:, None] == lax.broadcasted_iota(
        jnp.int32, (_BM, _NUM_CLASSES), 1)).astype(jnp.float32)
    emb = jnp.dot(onehot, tab_ref[...], preferred_element_type=jnp.float32)
    out_ref[:, _D:] = emb


def _tc_concat_gather(x2, labels, table):
    R = x2.shape[0]
    nb = R // _BM
    lab3 = labels.reshape(nb, 1, _BM)
    return pl.pallas_call(
        _tc_body,
        grid=(nb,),
        in_specs=[
            pl.BlockSpec((1, 1, _BM), lambda i: (i, 0, 0)),
            pl.BlockSpec((_BM, _D), lambda i: (i, 0)),
            pl.BlockSpec((_NUM_CLASSES, _D), lambda i: (0, 0)),
        ],
        out_specs=pl.BlockSpec((_BM, 2 * _D), lambda i: (i, 0)),
        out_shape=jax.ShapeDtypeStruct((R, 2 * _D), jnp.float32),
    )(lab3, x2, table)


def kernel(x, sem_labels, table, bbox):
    B, N, C = x.shape
    x2 = x.reshape(B * N, C)
    labels = sem_labels.reshape(B * N).astype(jnp.int32)
    out = _tc_concat_gather(x2, labels, table)
    return out.reshape(B, N, 2 * C)


# speedup vs baseline: 2.9291x; 1.3362x over previous
import functools

import jax
import jax.numpy as jnp
from jax import lax
from jax.experimental import pallas as pl
from jax.experimental.pallas import tpu as pltpu

_NUM_CLASSES = 150
_D = 768
_BM = 512


def _tc_body(lab_ref, x_ref, tab_ref, out_ref):
    out_ref[:, : _D] = x_ref[...]
    labels = lab_ref[0, 0, :]
    onehot = (labels[:, None] == lax.broadcasted_iota(
        jnp.int32, (_BM, _NUM_CLASSES), 1)).astype(jnp.float32)
    emb = jnp.dot(onehot, tab_ref[...], preferred_element_type=jnp.float32)
    out_ref[:, _D:] = emb


def _tc_concat_gather(x2, labels, table):
    R = x2.shape[0]
    nb = R // _BM
    lab3 = labels.reshape(nb, 1, _BM)
    return pl.pallas_call(
        _tc_body,
        grid=(nb,),
        in_specs=[
            pl.BlockSpec((1, 1, _BM), lambda i: (i, 0, 0)),
            pl.BlockSpec((_BM, _D), lambda i: (i, 0)),
            pl.BlockSpec((_NUM_CLASSES, _D), lambda i: (0, 0)),
        ],
        out_specs=pl.BlockSpec((_BM, 2 * _D), lambda i: (i, 0)),
        out_shape=jax.ShapeDtypeStruct((R, 2 * _D), jnp.float32),
    )(lab3, x2, table)


def kernel(x, sem_labels, table, bbox):
    B, N, C = x.shape
    x2 = x.reshape(B * N, C)
    labels = sem_labels.reshape(B * N).astype(jnp.int32)
    out = _tc_concat_gather(x2, labels, table)
    return out.reshape(B, N, 2 * C)


# speedup vs baseline: 3.2134x; 1.0971x over previous
import functools

import jax
import jax.numpy as jnp
from jax import lax
from jax.experimental import pallas as pl
from jax.experimental.pallas import tpu as pltpu

_NUM_CLASSES = 150
_D = 768
_BM = 1024


def _tc_body(lab_ref, x_ref, tab_ref, out_ref):
    out_ref[:, : _D] = x_ref[...]
    labels = lab_ref[0, 0, :]
    onehot = (labels[:, None] == lax.broadcasted_iota(
        jnp.int32, (_BM, _NUM_CLASSES), 1)).astype(jnp.float32)
    emb = jnp.dot(onehot, tab_ref[...], preferred_element_type=jnp.float32)
    out_ref[:, _D:] = emb


def _tc_concat_gather(x2, labels, table):
    R = x2.shape[0]
    nb = R // _BM
    lab3 = labels.reshape(nb, 1, _BM)
    return pl.pallas_call(
        _tc_body,
        grid=(nb,),
        in_specs=[
            pl.BlockSpec((1, 1, _BM), lambda i: (i, 0, 0)),
            pl.BlockSpec((_BM, _D), lambda i: (i, 0)),
            pl.BlockSpec((_NUM_CLASSES, _D), lambda i: (0, 0)),
        ],
        out_specs=pl.BlockSpec((_BM, 2 * _D), lambda i: (i, 0)),
        out_shape=jax.ShapeDtypeStruct((R, 2 * _D), jnp.float32),
    )(lab3, x2, table)


def kernel(x, sem_labels, table, bbox):
    B, N, C = x.shape
    x2 = x.reshape(B * N, C)
    labels = sem_labels.reshape(B * N).astype(jnp.int32)
    out = _tc_concat_gather(x2, labels, table)
    return out.reshape(B, N, 2 * C)


# speedup vs baseline: 3.3738x; 1.0499x over previous
import functools

import jax
import jax.numpy as jnp
from jax import lax
from jax.experimental import pallas as pl
from jax.experimental.pallas import tpu as pltpu

_NUM_CLASSES = 150
_D = 768
_BM = 2048


def _tc_body(lab_ref, x_ref, tab_ref, out_ref):
    out_ref[:, : _D] = x_ref[...]
    labels = lab_ref[0, 0, :]
    onehot = (labels[:, None] == lax.broadcasted_iota(
        jnp.int32, (_BM, _NUM_CLASSES), 1)).astype(jnp.float32)
    emb = jnp.dot(onehot, tab_ref[...], preferred_element_type=jnp.float32)
    out_ref[:, _D:] = emb


def _tc_concat_gather(x2, labels, table):
    R = x2.shape[0]
    nb = R // _BM
    lab3 = labels.reshape(nb, 1, _BM)
    return pl.pallas_call(
        _tc_body,
        grid=(nb,),
        in_specs=[
            pl.BlockSpec((1, 1, _BM), lambda i: (i, 0, 0)),
            pl.BlockSpec((_BM, _D), lambda i: (i, 0)),
            pl.BlockSpec((_NUM_CLASSES, _D), lambda i: (0, 0)),
        ],
        out_specs=pl.BlockSpec((_BM, 2 * _D), lambda i: (i, 0)),
        out_shape=jax.ShapeDtypeStruct((R, 2 * _D), jnp.float32),
    )(lab3, x2, table)


def kernel(x, sem_labels, table, bbox):
    B, N, C = x.shape
    x2 = x.reshape(B * N, C)
    labels = sem_labels.reshape(B * N).astype(jnp.int32)
    out = _tc_concat_gather(x2, labels, table)
    return out.reshape(B, N, 2 * C)
